# baseline (speedup 1.0000x reference)
import numpy as np
import jax
import jax.numpy as jnp
from jax.experimental import pallas as pl
from jax.experimental.pallas import tpu as pltpu

_VMEM_LIMIT = 48 * 1024 * 1024
_TARGET_M = 4096
_MERGE_MAX = 256


def _taps(n_in, n_out):
    if n_in == 1:
        return [(0, 0.0)] * n_out
    s = (n_in - 1) / (n_out - 1)
    out = []
    for i in range(n_out):
        src = i * s
        lo = min(int(np.floor(src)), n_in - 2)
        out.append((lo, float(src - lo)))
    return out


def _upmat(n_out, n_in):
    m = np.zeros((n_out, n_in), np.float32)
    for i, (lo, fr) in enumerate(_taps(n_in, n_out)):
        m[i, lo] += 1.0 - fr
        if fr:
            m[i, lo + 1] += fr
    return m


def _row_tiles(S, W):
    R = min(S, max(1, _TARGET_M // W))
    tiles = []
    r0 = 0
    while r0 < S:
        tiles.append((r0, min(R, S - r0)))
        r0 += R
    return tiles


def _const(shape):
    nd = len(shape)
    return pl.BlockSpec(tuple(shape), lambda n, _nd=nd: (0,) * _nd)


def _zero_trip_halo(Xs, B, Hp):
    X0, X1, X2 = Xs
    S, W, C = X1.shape
    zrow = jnp.zeros((W, C), X1.dtype)
    for X in Xs:
        for b in range(B):
            X[b * Hp] = zrow
            X[b * Hp + Hp - 1] = zrow
    zcol = jnp.zeros((S, 1, C), X1.dtype)
    X0[:, 0:1, :] = zcol
    X2[:, W - 1:W, :] = zcol


def _rezero_trip(Xs, B, Hp):
    if B == 1:
        return
    W, C = Xs[1].shape[1], Xs[1].shape[2]
    zrow = jnp.zeros((W, C), Xs[1].dtype)
    for X in Xs:
        for b in range(B - 1):
            X[b * Hp + Hp - 1] = zrow
            X[(b + 1) * Hp] = zrow


def _put_trip(Xs, row0, v, c0=0):
    X0, X1, X2 = Xs
    R, W, C = v.shape
    X1[row0:row0 + R, :, c0:c0 + C] = v
    X0[row0:row0 + R, 1:W, c0:c0 + C] = v[:, :W - 1, :]
    X2[row0:row0 + R, 0:W - 1, c0:c0 + C] = v[:, 1:, :]


def _stage_in(Xs, x_ref, B, H, c0=0):
    Hp = H + 2
    for b in range(B):
        _put_trip(Xs, b * Hp + 1, x_ref[b].astype(Xs[1].dtype), c0)


def _stage_pool(Xs, x_ref, B, Ho):
    Hp = Ho + 2
    C = Xs[1].shape[2]
    for b in range(B):
        m = jnp.maximum(x_ref[b, :, 0], x_ref[b, :, 1])
        _put_trip(Xs, b * Hp + 1,
                  jnp.maximum(m[:, :, :C], m[:, :, C:]).astype(Xs[1].dtype))


def _stage_up(Xs, x_ref, cm_ref, tbuf, tub2, B, h, c0=0):
    Ho = 2 * h
    Hp = Ho + 2
    for b in range(B):
        for i in range(h):
            tbuf[i] = jnp.dot(cm_ref[...], x_ref[b, i].astype(jnp.float32),
                              preferred_element_type=jnp.float32)
        for ho, (lo, fr) in enumerate(_taps(h, Ho)):
            if fr == 0.0:
                row = tbuf[lo]
            else:
                row = (1.0 - fr) * tbuf[lo] + fr * tbuf[lo + 1]
            tub2[ho] = row.astype(tub2.dtype)
        _put_trip(Xs, b * Hp + 1, tub2[...], c0)


def _w_trip(Xs, c0=0):
    W = Xs[1].shape[1]
    def wfn(r0, R, vals):
        _put_trip(Xs, 1 + r0, vals.reshape(R, W, vals.shape[-1]).astype(Xs[1].dtype),
                  c0)
    return wfn


def _w_plain(dst):
    W = dst.shape[1]
    def wfn(r0, R, vals):
        dst[r0:r0 + R, :, :] = vals.reshape(R, W, vals.shape[-1]).astype(dst.dtype)
    return wfn


def _w_stack(dst):
    W = dst.shape[1]
    def wfn(r0, R, vals):
        dst[1 + r0:1 + r0 + R, :, :] = (
            vals.reshape(R, W, vals.shape[-1]).astype(dst.dtype))
    return wfn


def _w_out(o_ref):
    W = o_ref.shape[2]
    def wfn(r0, R, vals):
        o_ref[0, r0:r0 + R, :, :] = (
            vals.reshape(R, W, vals.shape[-1]).astype(o_ref.dtype))
    return wfn


def _w_logits(o_ref, wo_ref, bo_ref):
    W = o_ref.shape[2]
    ncls = o_ref.shape[3]
    def wfn(r0, R, vals):
        lg = jnp.dot(vals.astype(jnp.bfloat16), wo_ref[0],
                     preferred_element_type=jnp.float32) + bo_ref[...]
        o_ref[0, r0:r0 + R, :, :] = lg.reshape(R, W, ncls).astype(o_ref.dtype)
    return wfn


def _conv3(srcs, w_refs, b_ref, relu, S, W, write_fn):
    bias = b_ref[...]
    cout = w_refs[0].shape[-1]
    for r0, R in _row_tiles(S, W):
        m = R * W
        acc = jnp.broadcast_to(bias, (m, cout))
        for Xs, wr in zip(srcs, w_refs):
            ci = Xs[1].shape[-1]
            for dy in range(3):
                for dx in range(3):
                    xop = Xs[dx][r0 + dy:r0 + dy + R]
                    acc = acc + jnp.dot(xop.reshape(m, ci), wr[dy * 3 + dx],
                                        preferred_element_type=jnp.float32)
        if relu:
            acc = jnp.maximum(acc, 0.0)
        write_fn(r0, R, acc)


def _conv1(srcs, w_refs, b_ref, relu, S, W, write_fn):
    bias = b_ref[...]
    cout = w_refs[0].shape[-1]
    for r0, R in _row_tiles(S, W):
        m = R * W
        acc = jnp.broadcast_to(bias, (m, cout))
        for (buf, off), wr in zip(srcs, w_refs):
            ci = buf.shape[-1]
            xop = buf[r0 + off:r0 + off + R]
            acc = acc + jnp.dot(xop.reshape(m, ci), wr[0],
                                preferred_element_type=jnp.float32)
        if relu:
            acc = jnp.maximum(acc, 0.0)
        write_fn(r0, R, acc)


def _copy_out(o_ref, src, B, H):
    Hp = H + 2
    for b in range(B):
        o_ref[b] = src[b * Hp + 1:b * Hp + 1 + H, :, :].astype(o_ref.dtype)


def _make_stem(H, W):
    S = H
    Hp = H + 2

    def body(x_ref,
             r1_w1, r1_b1, r1_w2x, r1_w2c, r1_b2, r1_w3x, r1_w3c1, r1_w3c2, r1_b3,
             r2_w1, r2_b1, r2_w2x, r2_w2c, r2_b2, r2_w3x, r2_w3c1, r2_w3c2, r2_b3,
             i_w1, i_b1, i_w2, i_b2,
             o_ref, *scr):
        px = scr[0:3]
        pa = scr[3:6]
        c2b = scr[6]
        pr = scr[7:10]
        pmid = scr[10:13]
        for Xs in (px, pa, pr, pmid):
            _zero_trip_halo(Xs, 1, Hp)
        _stage_in(px, x_ref, 1, H)
        c = pr[1].shape[-1] // 2
        for (w1, b1, w2x, w2c, b2, w3x, w3c1, w3c2, b3, coff) in (
                (r1_w1, r1_b1, r1_w2x, r1_w2c, r1_b2, r1_w3x, r1_w3c1, r1_w3c2,
                 r1_b3, c),
                (r2_w1, r2_b1, r2_w2x, r2_w2c, r2_b2, r2_w3x, r2_w3c1, r2_w3c2,
                 r2_b3, 0)):
            _conv3([px], [w1], b1, True, S, W, _w_trip(pa))
            _conv3([px, pa], [w2x, w2c], b2, True, S, W, _w_plain(c2b))
            _conv1([(px[1], 1), (pa[1], 1), (c2b, 0)], [w3x, w3c1, w3c2], b3,
                   False, S, W, _w_trip(pr, coff))
        _conv3([pr], [i_w1], i_b1, True, S, W, _w_trip(pmid))
        _conv3([pmid], [i_w2], i_b2, True, S, W, _w_out(o_ref))
    return body


def _stem_call(x, w):
    N, H, W, Cin = x.shape
    c_rdn = int(w["rdn1_c3_w0"].shape[-1])
    c_inc = int(w["inc_c2_w0"].shape[-1])
    names = ["rdn1_c1_w0", "rdn1_c1_b", "rdn1_c2_w0", "rdn1_c2_w1", "rdn1_c2_b",
             "rdn1_c3_w0", "rdn1_c3_w1", "rdn1_c3_w2", "rdn1_c3_b",
             "rdn2_c1_w0", "rdn2_c1_b", "rdn2_c2_w0", "rdn2_c2_w1", "rdn2_c2_b",
             "rdn2_c3_w0", "rdn2_c3_w1", "rdn2_c3_w2", "rdn2_c3_b"]
    i_w1 = jnp.concatenate([w["inc_c1_w0"], w["inc_c1_w1"]], axis=1)
    args = ([x] + [w[k] for k in names]
            + [i_w1, w["inc_c1_b"], w["inc_c2_w0"], w["inc_c2_b"]])
    in_specs = ([pl.BlockSpec((1, H, W, Cin), lambda n: (n, 0, 0, 0))]
                + [_const(a.shape) for a in args[1:]])
    Hp = H + 2

    def trip(C):
        return [pltpu.VMEM((Hp, W, C), jnp.bfloat16) for _ in range(3)]

    scratch = (trip(Cin) + trip(c_rdn) + [pltpu.VMEM((H, W, c_rdn), jnp.bfloat16)]
               + trip(2 * c_rdn) + trip(c_inc))
    return pl.pallas_call(
        _make_stem(H, W),
        out_shape=jax.ShapeDtypeStruct((N, H, W, c_inc), jnp.bfloat16),
        grid_spec=pltpu.PrefetchScalarGridSpec(
            num_scalar_prefetch=0, grid=(N,), in_specs=in_specs,
            out_specs=pl.BlockSpec((1, H, W, c_inc), lambda n: (n, 0, 0, 0)),
            scratch_shapes=scratch),
        compiler_params=pltpu.CompilerParams(
            dimension_semantics=("parallel",), vmem_limit_bytes=_VMEM_LIMIT),
    )(*args)


def _make_down(B, Ho, Wo):
    Hp = Ho + 2
    S = B * Hp - 2

    def body(xp_ref, w1, b1, w2, b2, o_ref, *scr):
        pin = scr[0:3]
        pmid = scr[3:6]
        pout = scr[6]
        for Xs in (pin, pmid):
            _zero_trip_halo(Xs, B, Hp)
        _stage_pool(pin, xp_ref, B, Ho)
        _conv3([pin], [w1], b1, True, S, Wo, _w_trip(pmid))
        _rezero_trip(pmid, B, Hp)
        if B == 1:
            _conv3([pmid], [w2], b2, True, S, Wo, _w_out(o_ref))
        else:
            _conv3([pmid], [w2], b2, True, S, Wo, _w_stack(pout))
            _copy_out(o_ref, pout, B, Ho)
    return body


def _down_call(x, w1, b1, w2, b2, B):
    N, H, W, Cin = x.shape
    Ho, Wo = H // 2, W // 2
    Cmid = int(w1.shape[-1])
    Cout = int(w2.shape[-1])
    xp = x.reshape(N, Ho, 2, Wo, 2 * Cin)
    args = [xp, w1, b1, w2, b2]
    in_specs = ([pl.BlockSpec((B, Ho, 2, Wo, 2 * Cin),
                              lambda n: (n, 0, 0, 0, 0))]
                + [_const(a.shape) for a in args[1:]])
    Hp = Ho + 2
    scratch = ([pltpu.VMEM((B * Hp, Wo, Cin), jnp.bfloat16) for _ in range(3)]
               + [pltpu.VMEM((B * Hp, Wo, Cmid), jnp.bfloat16) for _ in range(3)]
               + [pltpu.VMEM((B * Hp, Wo, Cout), jnp.bfloat16)])
    return pl.pallas_call(
        _make_down(B, Ho, Wo),
        out_shape=jax.ShapeDtypeStruct((N, Ho, Wo, Cout), jnp.bfloat16),
        grid_spec=pltpu.PrefetchScalarGridSpec(
            num_scalar_prefetch=0, grid=(N // B,), in_specs=in_specs,
            out_specs=pl.BlockSpec((B, Ho, Wo, Cout), lambda n: (n, 0, 0, 0)),
            scratch_shapes=scratch),
        compiler_params=pltpu.CompilerParams(
            dimension_semantics=("parallel",), vmem_limit_bytes=_VMEM_LIMIT),
    )(*args)


def _make_up(B, Ho, Wo, with_outc, merged, Cskip):
    Hp = Ho + 2
    S = B * Hp - 2
    h = Ho // 2

    def body(*refs):
        if with_outc:
            (cm_ref, xd_ref, xs_ref, w1s, w1u, b1, w2, b2, wo, bo,
             o_ref) = refs[:11]
            scr = refs[11:]
        else:
            (cm_ref, xd_ref, xs_ref, w1s, w1u, b1, w2, b2, o_ref) = refs[:9]
            scr = refs[9:]
        if merged:
            pcat = scr[0:3]
            tbuf, tub2 = scr[3], scr[4]
            pmid = scr[5:8]
            pout = scr[8]
            _zero_trip_halo(pcat, B, Hp)
            _zero_trip_halo(pmid, B, Hp)
            _stage_in(pcat, xs_ref, B, Ho)
            _stage_up(pcat, xd_ref, cm_ref, tbuf, tub2, B, h, Cskip)
            srcs, ws = [pcat], [w1s]
        else:
            pskip = scr[0:3]
            pup = scr[3:6]
            tbuf, tub2 = scr[6], scr[7]
            pmid = scr[8:11]
            pout = scr[11]
            for Xs in (pskip, pup, pmid):
                _zero_trip_halo(Xs, B, Hp)
            _stage_in(pskip, xs_ref, B, Ho)
            _stage_up(pup, xd_ref, cm_ref, tbuf, tub2, B, h)
            srcs, ws = [pskip, pup], [w1s, w1u]
        _conv3(srcs, ws, b1, True, S, Wo, _w_trip(pmid))
        _rezero_trip(pmid, B, Hp)
        if with_outc:
            _conv3([pmid], [w2], b2, True, S, Wo, _w_logits(o_ref, wo, bo))
        elif B == 1:
            _conv3([pmid], [w2], b2, True, S, Wo, _w_out(o_ref))
        else:
            _conv3([pmid], [w2], b2, True, S, Wo, _w_stack(pout))
            _copy_out(o_ref, pout, B, Ho)
    return body


def _up_call(x_dec, x_skip, w1s, w1u, b1, w2, b2, B, wo=None, bo=None):
    N, h, wi, Cdec = x_dec.shape
    _, Ho, Wo, Cskip = x_skip.shape
    Cmid = int(w1s.shape[-1])
    Cout = int(w2.shape[-1])
    with_outc = wo is not None
    n_out = int(wo.shape[-1]) if with_outc else Cout
    out_dtype = jnp.float32 if with_outc else jnp.bfloat16
    cm = jnp.asarray(_upmat(Wo, wi), jnp.float32)
    merged = (Cskip + Cdec) <= _MERGE_MAX
    if merged:
        w1s = jnp.concatenate([w1s, w1u], axis=1)
    args = [cm, x_dec, x_skip, w1s, w1u, b1, w2, b2]
    if with_outc:
        args += [wo, bo]
    in_specs = ([_const(cm.shape),
                 pl.BlockSpec((B, h, wi, Cdec), lambda n: (n, 0, 0, 0)),
                 pl.BlockSpec((B, Ho, Wo, Cskip), lambda n: (n, 0, 0, 0))]
                + [_const(a.shape) for a in args[3:]])
    Hp = Ho + 2
    if merged:
        srcbufs = [pltpu.VMEM((B * Hp, Wo, Cskip + Cdec), jnp.bfloat16)
                   for _ in range(3)]
    else:
        srcbufs = ([pltpu.VMEM((B * Hp, Wo, Cskip), jnp.bfloat16)
                    for _ in range(3)]
                   + [pltpu.VMEM((B * Hp, Wo, Cdec), jnp.bfloat16)
                      for _ in range(3)])
    scratch = (srcbufs
               + [pltpu.VMEM((h, Wo, Cdec), jnp.float32)]
               + [pltpu.VMEM((Ho, Wo, Cdec), jnp.bfloat16)]
               + [pltpu.VMEM((B * Hp, Wo, Cmid), jnp.bfloat16) for _ in range(3)]
               + [pltpu.VMEM((B * Hp, Wo, Cout), jnp.bfloat16)])
    return pl.pallas_call(
        _make_up(B, Ho, Wo, with_outc, merged, Cskip),
        out_shape=jax.ShapeDtypeStruct((N, Ho, Wo, n_out), out_dtype),
        grid_spec=pltpu.PrefetchScalarGridSpec(
            num_scalar_prefetch=0, grid=(N // B,), in_specs=in_specs,
            out_specs=pl.BlockSpec((B, Ho, Wo, n_out), lambda n: (n, 0, 0, 0)),
            scratch_shapes=scratch),
        compiler_params=pltpu.CompilerParams(
            dimension_semantics=("parallel",), vmem_limit_bytes=_VMEM_LIMIT),
    )(*args)


def kernel(x, rdn1_c1_w0, rdn1_c1_b, rdn1_c2_w0, rdn1_c2_w1, rdn1_c2_b, rdn1_c3_w0, rdn1_c3_w1, rdn1_c3_w2, rdn1_c3_b, rdn2_c1_w0, rdn2_c1_b, rdn2_c2_w0, rdn2_c2_w1, rdn2_c2_b, rdn2_c3_w0, rdn2_c3_w1, rdn2_c3_w2, rdn2_c3_b, inc_c1_w0, inc_c1_w1, inc_c1_b, inc_c2_w0, inc_c2_b, down1_c1_w0, down1_c1_b, down1_c2_w0, down1_c2_b, down2_c1_w0, down2_c1_b, down2_c2_w0, down2_c2_b, down3_c1_w0, down3_c1_b, down3_c2_w0, down3_c2_b, down4_c1_w0, down4_c1_b, down4_c2_w0, down4_c2_b, up1_c1_w0, up1_c1_w1, up1_c1_b, up1_c2_w0, up1_c2_b, up2_c1_w0, up2_c1_w1, up2_c1_b, up2_c2_w0, up2_c2_b, up3_c1_w0, up3_c1_w1, up3_c1_b, up3_c2_w0, up3_c2_b, up4_c1_w0, up4_c1_w1, up4_c1_b, up4_c2_w0, up4_c2_b, outc_w0, outc_b):
    stem_w = dict(
        rdn1_c1_w0=rdn1_c1_w0, rdn1_c1_b=rdn1_c1_b, rdn1_c2_w0=rdn1_c2_w0,
        rdn1_c2_w1=rdn1_c2_w1, rdn1_c2_b=rdn1_c2_b, rdn1_c3_w0=rdn1_c3_w0,
        rdn1_c3_w1=rdn1_c3_w1, rdn1_c3_w2=rdn1_c3_w2, rdn1_c3_b=rdn1_c3_b,
        rdn2_c1_w0=rdn2_c1_w0, rdn2_c1_b=rdn2_c1_b, rdn2_c2_w0=rdn2_c2_w0,
        rdn2_c2_w1=rdn2_c2_w1, rdn2_c2_b=rdn2_c2_b, rdn2_c3_w0=rdn2_c3_w0,
        rdn2_c3_w1=rdn2_c3_w1, rdn2_c3_w2=rdn2_c3_w2, rdn2_c3_b=rdn2_c3_b,
        inc_c1_w0=inc_c1_w0, inc_c1_w1=inc_c1_w1, inc_c1_b=inc_c1_b,
        inc_c2_w0=inc_c2_w0, inc_c2_b=inc_c2_b)
    xh = jnp.transpose(x, (0, 2, 3, 1)).astype(jnp.bfloat16)
    x1 = _stem_call(xh, stem_w)
    x2 = _down_call(x1, down1_c1_w0, down1_c1_b, down1_c2_w0, down1_c2_b, B=1)
    x3 = _down_call(x2, down2_c1_w0, down2_c1_b, down2_c2_w0, down2_c2_b, B=4)
    x4 = _down_call(x3, down3_c1_w0, down3_c1_b, down3_c2_w0, down3_c2_b, B=8)
    x5 = _down_call(x4, down4_c1_w0, down4_c1_b, down4_c2_w0, down4_c2_b, B=16)
    y = _up_call(x5, x4, up1_c1_w0, up1_c1_w1, up1_c1_b, up1_c2_w0, up1_c2_b, B=8)
    y = _up_call(y, x3, up2_c1_w0, up2_c1_w1, up2_c1_b, up2_c2_w0, up2_c2_b, B=4)
    y = _up_call(y, x2, up3_c1_w0, up3_c1_w1, up3_c1_b, up3_c2_w0, up3_c2_b, B=1)
    logits = _up_call(y, x1, up4_c1_w0, up4_c1_w1, up4_c1_b, up4_c2_w0,
                      up4_c2_b, B=1, wo=outc_w0, bo=outc_b)
    return jnp.transpose(logits, (0, 3, 1, 2))


# speedup vs baseline: 1.2430x; 1.2430x over previous
import numpy as np
import jax
import jax.numpy as jnp
from jax.experimental import pallas as pl
from jax.experimental.pallas import tpu as pltpu

_VMEM_LIMIT = 48 * 1024 * 1024
_TARGET_M = 1024
_MERGE_MAX = 256


def _taps(n_in, n_out):
    if n_in == 1:
        return [(0, 0.0)] * n_out
    s = (n_in - 1) / (n_out - 1)
    out = []
    for i in range(n_out):
        src = i * s
        lo = min(int(np.floor(src)), n_in - 2)
        out.append((lo, float(src - lo)))
    return out


def _upmat(n_out, n_in):
    m = np.zeros((n_out, n_in), np.float32)
    for i, (lo, fr) in enumerate(_taps(n_in, n_out)):
        m[i, lo] += 1.0 - fr
        if fr:
            m[i, lo + 1] += fr
    return m


def _row_tiles(S, W):
    R = min(S, max(1, _TARGET_M // W))
    tiles = []
    r0 = 0
    while r0 < S:
        tiles.append((r0, min(R, S - r0)))
        r0 += R
    return tiles


def _const(shape):
    nd = len(shape)
    return pl.BlockSpec(tuple(shape), lambda n, _nd=nd: (0,) * _nd)


def _zero_trip_halo(Xs, B, Hp):
    X0, X1, X2 = Xs
    S, W, C = X1.shape
    zrow = jnp.zeros((W, C), X1.dtype)
    for X in Xs:
        for b in range(B):
            X[b * Hp] = zrow
            X[b * Hp + Hp - 1] = zrow
    zcol = jnp.zeros((S, 1, C), X1.dtype)
    X0[:, 0:1, :] = zcol
    X2[:, W - 1:W, :] = zcol


def _rezero_trip(Xs, B, Hp):
    if B == 1:
        return
    W, C = Xs[1].shape[1], Xs[1].shape[2]
    zrow = jnp.zeros((W, C), Xs[1].dtype)
    for X in Xs:
        for b in range(B - 1):
            X[b * Hp + Hp - 1] = zrow
            X[(b + 1) * Hp] = zrow


def _put_trip(Xs, row0, v, c0=0):
    X0, X1, X2 = Xs
    R, W, C = v.shape
    X1[row0:row0 + R, :, c0:c0 + C] = v
    X0[row0:row0 + R, 1:W, c0:c0 + C] = v[:, :W - 1, :]
    X2[row0:row0 + R, 0:W - 1, c0:c0 + C] = v[:, 1:, :]


def _stage_in(Xs, x_ref, B, H, c0=0):
    Hp = H + 2
    for b in range(B):
        _put_trip(Xs, b * Hp + 1, x_ref[b].astype(Xs[1].dtype), c0)


def _stage_pool(Xs, x_ref, B, Ho):
    Hp = Ho + 2
    C = Xs[1].shape[2]
    for b in range(B):
        m = jnp.maximum(x_ref[b, :, 0], x_ref[b, :, 1])
        _put_trip(Xs, b * Hp + 1,
                  jnp.maximum(m[:, :, :C], m[:, :, C:]).astype(Xs[1].dtype))


def _stage_up(Xs, x_ref, cm_ref, tbuf, tub2, B, h, c0=0):
    Ho = 2 * h
    Hp = Ho + 2
    for b in range(B):
        for i in range(h):
            tbuf[i] = jnp.dot(cm_ref[...], x_ref[b, i].astype(jnp.float32),
                              preferred_element_type=jnp.float32)
        for ho, (lo, fr) in enumerate(_taps(h, Ho)):
            if fr == 0.0:
                row = tbuf[lo]
            else:
                row = (1.0 - fr) * tbuf[lo] + fr * tbuf[lo + 1]
            tub2[ho] = row.astype(tub2.dtype)
        _put_trip(Xs, b * Hp + 1, tub2[...], c0)


def _w_trip(Xs, c0=0):
    W = Xs[1].shape[1]
    def wfn(r0, R, vals):
        _put_trip(Xs, 1 + r0, vals.reshape(R, W, vals.shape[-1]).astype(Xs[1].dtype),
                  c0)
    return wfn


def _w_plain(dst):
    W = dst.shape[1]
    def wfn(r0, R, vals):
        dst[r0:r0 + R, :, :] = vals.reshape(R, W, vals.shape[-1]).astype(dst.dtype)
    return wfn


def _w_stack(dst):
    W = dst.shape[1]
    def wfn(r0, R, vals):
        dst[1 + r0:1 + r0 + R, :, :] = (
            vals.reshape(R, W, vals.shape[-1]).astype(dst.dtype))
    return wfn


def _w_out(o_ref):
    W = o_ref.shape[2]
    def wfn(r0, R, vals):
        o_ref[0, r0:r0 + R, :, :] = (
            vals.reshape(R, W, vals.shape[-1]).astype(o_ref.dtype))
    return wfn


def _w_logits(o_ref, wo_ref, bo_ref):
    W = o_ref.shape[2]
    ncls = o_ref.shape[3]
    def wfn(r0, R, vals):
        lg = jnp.dot(vals.astype(jnp.bfloat16), wo_ref[0],
                     preferred_element_type=jnp.float32) + bo_ref[...]
        o_ref[0, r0:r0 + R, :, :] = lg.reshape(R, W, ncls).astype(o_ref.dtype)
    return wfn


def _conv3(srcs, w_refs, b_ref, relu, S, W, write_fn):
    bias = b_ref[...]
    cout = w_refs[0].shape[-1]
    for r0, R in _row_tiles(S, W):
        m = R * W
        acc = jnp.broadcast_to(bias, (m, cout))
        for Xs, wr in zip(srcs, w_refs):
            ci = Xs[1].shape[-1]
            for dy in range(3):
                for dx in range(3):
                    xop = Xs[dx][r0 + dy:r0 + dy + R]
                    acc = acc + jnp.dot(xop.reshape(m, ci), wr[dy * 3 + dx],
                                        preferred_element_type=jnp.float32)
        if relu:
            acc = jnp.maximum(acc, 0.0)
        write_fn(r0, R, acc)


def _conv1(srcs, w_refs, b_ref, relu, S, W, write_fn):
    bias = b_ref[...]
    cout = w_refs[0].shape[-1]
    for r0, R in _row_tiles(S, W):
        m = R * W
        acc = jnp.broadcast_to(bias, (m, cout))
        for (buf, off), wr in zip(srcs, w_refs):
            ci = buf.shape[-1]
            xop = buf[r0 + off:r0 + off + R]
            acc = acc + jnp.dot(xop.reshape(m, ci), wr[0],
                                preferred_element_type=jnp.float32)
        if relu:
            acc = jnp.maximum(acc, 0.0)
        write_fn(r0, R, acc)


def _copy_out(o_ref, src, B, H):
    Hp = H + 2
    for b in range(B):
        o_ref[b] = src[b * Hp + 1:b * Hp + 1 + H, :, :].astype(o_ref.dtype)


def _make_stem(H, W):
    S = H
    Hp = H + 2

    def body(x_ref,
             r1_w1, r1_b1, r1_w2x, r1_w2c, r1_b2, r1_w3x, r1_w3c1, r1_w3c2, r1_b3,
             r2_w1, r2_b1, r2_w2x, r2_w2c, r2_b2, r2_w3x, r2_w3c1, r2_w3c2, r2_b3,
             i_w1, i_b1, i_w2, i_b2,
             o_ref, *scr):
        px = scr[0:3]
        pa = scr[3:6]
        c2b = scr[6]
        pr = scr[7:10]
        pmid = scr[10:13]
        for Xs in (px, pa, pr, pmid):
            _zero_trip_halo(Xs, 1, Hp)
        _stage_in(px, x_ref, 1, H)
        c = pr[1].shape[-1] // 2
        for (w1, b1, w2x, w2c, b2, w3x, w3c1, w3c2, b3, coff) in (
                (r1_w1, r1_b1, r1_w2x, r1_w2c, r1_b2, r1_w3x, r1_w3c1, r1_w3c2,
                 r1_b3, c),
                (r2_w1, r2_b1, r2_w2x, r2_w2c, r2_b2, r2_w3x, r2_w3c1, r2_w3c2,
                 r2_b3, 0)):
            _conv3([px], [w1], b1, True, S, W, _w_trip(pa))
            _conv3([px, pa], [w2x, w2c], b2, True, S, W, _w_plain(c2b))
            _conv1([(px[1], 1), (pa[1], 1), (c2b, 0)], [w3x, w3c1, w3c2], b3,
                   False, S, W, _w_trip(pr, coff))
        _conv3([pr], [i_w1], i_b1, True, S, W, _w_trip(pmid))
        _conv3([pmid], [i_w2], i_b2, True, S, W, _w_out(o_ref))
    return body


def _stem_call(x, w):
    N, H, W, Cin = x.shape
    c_rdn = int(w["rdn1_c3_w0"].shape[-1])
    c_inc = int(w["inc_c2_w0"].shape[-1])
    names = ["rdn1_c1_w0", "rdn1_c1_b", "rdn1_c2_w0", "rdn1_c2_w1", "rdn1_c2_b",
             "rdn1_c3_w0", "rdn1_c3_w1", "rdn1_c3_w2", "rdn1_c3_b",
             "rdn2_c1_w0", "rdn2_c1_b", "rdn2_c2_w0", "rdn2_c2_w1", "rdn2_c2_b",
             "rdn2_c3_w0", "rdn2_c3_w1", "rdn2_c3_w2", "rdn2_c3_b"]
    i_w1 = jnp.concatenate([w["inc_c1_w0"], w["inc_c1_w1"]], axis=1)
    args = ([x] + [w[k] for k in names]
            + [i_w1, w["inc_c1_b"], w["inc_c2_w0"], w["inc_c2_b"]])
    in_specs = ([pl.BlockSpec((1, H, W, Cin), lambda n: (n, 0, 0, 0))]
                + [_const(a.shape) for a in args[1:]])
    Hp = H + 2

    def trip(C):
        return [pltpu.VMEM((Hp, W, C), jnp.bfloat16) for _ in range(3)]

    scratch = (trip(Cin) + trip(c_rdn) + [pltpu.VMEM((H, W, c_rdn), jnp.bfloat16)]
               + trip(2 * c_rdn) + trip(c_inc))
    return pl.pallas_call(
        _make_stem(H, W),
        out_shape=jax.ShapeDtypeStruct((N, H, W, c_inc), jnp.bfloat16),
        grid_spec=pltpu.PrefetchScalarGridSpec(
            num_scalar_prefetch=0, grid=(N,), in_specs=in_specs,
            out_specs=pl.BlockSpec((1, H, W, c_inc), lambda n: (n, 0, 0, 0)),
            scratch_shapes=scratch),
        compiler_params=pltpu.CompilerParams(
            dimension_semantics=("parallel",), vmem_limit_bytes=_VMEM_LIMIT),
    )(*args)


def _make_down(B, Ho, Wo):
    Hp = Ho + 2
    S = B * Hp - 2

    def body(xp_ref, w1, b1, w2, b2, o_ref, *scr):
        pin = scr[0:3]
        pmid = scr[3:6]
        pout = scr[6]
        for Xs in (pin, pmid):
            _zero_trip_halo(Xs, B, Hp)
        _stage_pool(pin, xp_ref, B, Ho)
        _conv3([pin], [w1], b1, True, S, Wo, _w_trip(pmid))
        _rezero_trip(pmid, B, Hp)
        if B == 1:
            _conv3([pmid], [w2], b2, True, S, Wo, _w_out(o_ref))
        else:
            _conv3([pmid], [w2], b2, True, S, Wo, _w_stack(pout))
            _copy_out(o_ref, pout, B, Ho)
    return body


def _down_call(x, w1, b1, w2, b2, B):
    N, H, W, Cin = x.shape
    Ho, Wo = H // 2, W // 2
    Cmid = int(w1.shape[-1])
    Cout = int(w2.shape[-1])
    xp = x.reshape(N, Ho, 2, Wo, 2 * Cin)
    args = [xp, w1, b1, w2, b2]
    in_specs = ([pl.BlockSpec((B, Ho, 2, Wo, 2 * Cin),
                              lambda n: (n, 0, 0, 0, 0))]
                + [_const(a.shape) for a in args[1:]])
    Hp = Ho + 2
    scratch = ([pltpu.VMEM((B * Hp, Wo, Cin), jnp.bfloat16) for _ in range(3)]
               + [pltpu.VMEM((B * Hp, Wo, Cmid), jnp.bfloat16) for _ in range(3)]
               + [pltpu.VMEM((B * Hp, Wo, Cout), jnp.bfloat16)])
    return pl.pallas_call(
        _make_down(B, Ho, Wo),
        out_shape=jax.ShapeDtypeStruct((N, Ho, Wo, Cout), jnp.bfloat16),
        grid_spec=pltpu.PrefetchScalarGridSpec(
            num_scalar_prefetch=0, grid=(N // B,), in_specs=in_specs,
            out_specs=pl.BlockSpec((B, Ho, Wo, Cout), lambda n: (n, 0, 0, 0)),
            scratch_shapes=scratch),
        compiler_params=pltpu.CompilerParams(
            dimension_semantics=("parallel",), vmem_limit_bytes=_VMEM_LIMIT),
    )(*args)


def _make_up(B, Ho, Wo, with_outc, merged, Cskip):
    Hp = Ho + 2
    S = B * Hp - 2
    h = Ho // 2

    def body(*refs):
        if with_outc:
            (cm_ref, xd_ref, xs_ref, w1s, w1u, b1, w2, b2, wo, bo,
             o_ref) = refs[:11]
            scr = refs[11:]
        else:
            (cm_ref, xd_ref, xs_ref, w1s, w1u, b1, w2, b2, o_ref) = refs[:9]
            scr = refs[9:]
        if merged:
            pcat = scr[0:3]
            tbuf, tub2 = scr[3], scr[4]
            pmid = scr[5:8]
            pout = scr[8]
            _zero_trip_halo(pcat, B, Hp)
            _zero_trip_halo(pmid, B, Hp)
            _stage_in(pcat, xs_ref, B, Ho)
            _stage_up(pcat, xd_ref, cm_ref, tbuf, tub2, B, h, Cskip)
            srcs, ws = [pcat], [w1s]
        else:
            pskip = scr[0:3]
            pup = scr[3:6]
            tbuf, tub2 = scr[6], scr[7]
            pmid = scr[8:11]
            pout = scr[11]
            for Xs in (pskip, pup, pmid):
                _zero_trip_halo(Xs, B, Hp)
            _stage_in(pskip, xs_ref, B, Ho)
            _stage_up(pup, xd_ref, cm_ref, tbuf, tub2, B, h)
            srcs, ws = [pskip, pup], [w1s, w1u]
        _conv3(srcs, ws, b1, True, S, Wo, _w_trip(pmid))
        _rezero_trip(pmid, B, Hp)
        if with_outc:
            _conv3([pmid], [w2], b2, True, S, Wo, _w_logits(o_ref, wo, bo))
        elif B == 1:
            _conv3([pmid], [w2], b2, True, S, Wo, _w_out(o_ref))
        else:
            _conv3([pmid], [w2], b2, True, S, Wo, _w_stack(pout))
            _copy_out(o_ref, pout, B, Ho)
    return body


def _up_call(x_dec, x_skip, w1s, w1u, b1, w2, b2, B, wo=None, bo=None):
    N, h, wi, Cdec = x_dec.shape
    _, Ho, Wo, Cskip = x_skip.shape
    Cmid = int(w1s.shape[-1])
    Cout = int(w2.shape[-1])
    with_outc = wo is not None
    n_out = int(wo.shape[-1]) if with_outc else Cout
    out_dtype = jnp.float32 if with_outc else jnp.bfloat16
    cm = jnp.asarray(_upmat(Wo, wi), jnp.float32)
    merged = (Cskip + Cdec) <= _MERGE_MAX
    if merged:
        w1s = jnp.concatenate([w1s, w1u], axis=1)
    args = [cm, x_dec, x_skip, w1s, w1u, b1, w2, b2]
    if with_outc:
        args += [wo, bo]
    in_specs = ([_const(cm.shape),
                 pl.BlockSpec((B, h, wi, Cdec), lambda n: (n, 0, 0, 0)),
                 pl.BlockSpec((B, Ho, Wo, Cskip), lambda n: (n, 0, 0, 0))]
                + [_const(a.shape) for a in args[3:]])
    Hp = Ho + 2
    if merged:
        srcbufs = [pltpu.VMEM((B * Hp, Wo, Cskip + Cdec), jnp.bfloat16)
                   for _ in range(3)]
    else:
        srcbufs = ([pltpu.VMEM((B * Hp, Wo, Cskip), jnp.bfloat16)
                    for _ in range(3)]
                   + [pltpu.VMEM((B * Hp, Wo, Cdec), jnp.bfloat16)
                      for _ in range(3)])
    scratch = (srcbufs
               + [pltpu.VMEM((h, Wo, Cdec), jnp.float32)]
               + [pltpu.VMEM((Ho, Wo, Cdec), jnp.bfloat16)]
               + [pltpu.VMEM((B * Hp, Wo, Cmid), jnp.bfloat16) for _ in range(3)]
               + [pltpu.VMEM((B * Hp, Wo, Cout), jnp.bfloat16)])
    return pl.pallas_call(
        _make_up(B, Ho, Wo, with_outc, merged, Cskip),
        out_shape=jax.ShapeDtypeStruct((N, Ho, Wo, n_out), out_dtype),
        grid_spec=pltpu.PrefetchScalarGridSpec(
            num_scalar_prefetch=0, grid=(N // B,), in_specs=in_specs,
            out_specs=pl.BlockSpec((B, Ho, Wo, n_out), lambda n: (n, 0, 0, 0)),
            scratch_shapes=scratch),
        compiler_params=pltpu.CompilerParams(
            dimension_semantics=("parallel",), vmem_limit_bytes=_VMEM_LIMIT),
    )(*args)


def kernel(x, rdn1_c1_w0, rdn1_c1_b, rdn1_c2_w0, rdn1_c2_w1, rdn1_c2_b, rdn1_c3_w0, rdn1_c3_w1, rdn1_c3_w2, rdn1_c3_b, rdn2_c1_w0, rdn2_c1_b, rdn2_c2_w0, rdn2_c2_w1, rdn2_c2_b, rdn2_c3_w0, rdn2_c3_w1, rdn2_c3_w2, rdn2_c3_b, inc_c1_w0, inc_c1_w1, inc_c1_b, inc_c2_w0, inc_c2_b, down1_c1_w0, down1_c1_b, down1_c2_w0, down1_c2_b, down2_c1_w0, down2_c1_b, down2_c2_w0, down2_c2_b, down3_c1_w0, down3_c1_b, down3_c2_w0, down3_c2_b, down4_c1_w0, down4_c1_b, down4_c2_w0, down4_c2_b, up1_c1_w0, up1_c1_w1, up1_c1_b, up1_c2_w0, up1_c2_b, up2_c1_w0, up2_c1_w1, up2_c1_b, up2_c2_w0, up2_c2_b, up3_c1_w0, up3_c1_w1, up3_c1_b, up3_c2_w0, up3_c2_b, up4_c1_w0, up4_c1_w1, up4_c1_b, up4_c2_w0, up4_c2_b, outc_w0, outc_b):
    stem_w = dict(
        rdn1_c1_w0=rdn1_c1_w0, rdn1_c1_b=rdn1_c1_b, rdn1_c2_w0=rdn1_c2_w0,
        rdn1_c2_w1=rdn1_c2_w1, rdn1_c2_b=rdn1_c2_b, rdn1_c3_w0=rdn1_c3_w0,
        rdn1_c3_w1=rdn1_c3_w1, rdn1_c3_w2=rdn1_c3_w2, rdn1_c3_b=rdn1_c3_b,
        rdn2_c1_w0=rdn2_c1_w0, rdn2_c1_b=rdn2_c1_b, rdn2_c2_w0=rdn2_c2_w0,
        rdn2_c2_w1=rdn2_c2_w1, rdn2_c2_b=rdn2_c2_b, rdn2_c3_w0=rdn2_c3_w0,
        rdn2_c3_w1=rdn2_c3_w1, rdn2_c3_w2=rdn2_c3_w2, rdn2_c3_b=rdn2_c3_b,
        inc_c1_w0=inc_c1_w0, inc_c1_w1=inc_c1_w1, inc_c1_b=inc_c1_b,
        inc_c2_w0=inc_c2_w0, inc_c2_b=inc_c2_b)
    xh = jnp.transpose(x, (0, 2, 3, 1)).astype(jnp.bfloat16)
    x1 = _stem_call(xh, stem_w)
    x2 = _down_call(x1, down1_c1_w0, down1_c1_b, down1_c2_w0, down1_c2_b, B=1)
    x3 = _down_call(x2, down2_c1_w0, down2_c1_b, down2_c2_w0, down2_c2_b, B=4)
    x4 = _down_call(x3, down3_c1_w0, down3_c1_b, down3_c2_w0, down3_c2_b, B=8)
    x5 = _down_call(x4, down4_c1_w0, down4_c1_b, down4_c2_w0, down4_c2_b, B=16)
    y = _up_call(x5, x4, up1_c1_w0, up1_c1_w1, up1_c1_b, up1_c2_w0, up1_c2_b, B=8)
    y = _up_call(y, x3, up2_c1_w0, up2_c1_w1, up2_c1_b, up2_c2_w0, up2_c2_b, B=4)
    y = _up_call(y, x2, up3_c1_w0, up3_c1_w1, up3_c1_b, up3_c2_w0, up3_c2_b, B=1)
    logits = _up_call(y, x1, up4_c1_w0, up4_c1_w1, up4_c1_b, up4_c2_w0,
                      up4_c2_b, B=1, wo=outc_w0, bo=outc_b)
    return jnp.transpose(logits, (0, 3, 1, 2))
